# baseline (speedup 1.0000x reference)
import functools

import jax
import jax.numpy as jnp
from jax import lax
from jax.experimental import pallas as pl
from jax.experimental.pallas import tpu as pltpu
from jax.experimental.pallas import tpu_sc as plsc

N_ATOMS = 100000
N_TYPES = 50
CHUNK = 1024
N_FULL = N_ATOMS // CHUNK
N_BLOCKS = pl.cdiv(N_ATOMS, CHUNK)
GROUPS = CHUNK // 16
NW = 32


@functools.partial(
    pl.kernel,
    out_type=jax.ShapeDtypeStruct((N_TYPES, N_ATOMS), jnp.float32),
    mesh=plsc.VectorSubcoreMesh(core_axis_name="c", subcore_axis_name="s"),
    scratch_types=[
        pltpu.VMEM((CHUNK,), jnp.int32),
        pltpu.VMEM((CHUNK,), jnp.int32),
        pltpu.VMEM((N_TYPES, CHUNK), jnp.float32),
        pltpu.VMEM((N_TYPES, CHUNK), jnp.float32),
        pltpu.SemaphoreType.DMA,
        pltpu.SemaphoreType.DMA,
    ],
    compiler_params=pltpu.CompilerParams(
        needs_layout_passes=False, skip_device_barrier=True
    ),
)
def _onehot_sc(types_hbm, zeros_hbm, out_hbm, t0, t1, b0, b1, s0, s1):
    wid = lax.axis_index("s") * 2 + lax.axis_index("c")
    ones16 = jnp.ones((16,), jnp.float32)
    zeros16 = jnp.zeros((16,), jnp.float32)
    iota16 = lax.iota(jnp.int32, 16)
    tvs, bufs, sems = [t0, t1], [b0, b1], [s0, s1]

    pltpu.sync_copy(zeros_hbm, b0)
    pltpu.sync_copy(zeros_hbm, b1)

    for k in range(4):
        c = wid + k * NW
        tv, buf, sem = tvs[k % 2], bufs[k % 2], sems[k % 2]

        @pl.when(c < N_FULL)
        def _(tv=tv, buf=buf, sem=sem, c=c, k=k):
            base = c * CHUNK
            if k >= 2:
                pltpu.make_async_copy(
                    buf, out_hbm.at[:, pl.ds((c - 2 * NW) * CHUNK, CHUNK)], sem
                ).wait()
                for g in range(GROUPS):
                    t = tv[pl.ds(g * 16, 16)]
                    plsc.store_scatter(buf, [t, iota16 + g * 16], zeros16)
            pltpu.sync_copy(types_hbm.at[pl.ds(base, CHUNK)], tv)
            for g in range(GROUPS):
                t = tv[pl.ds(g * 16, 16)]
                plsc.store_scatter(buf, [t, iota16 + g * 16], ones16)
            pltpu.async_copy(buf, out_hbm.at[:, pl.ds(base, CHUNK)], sem)

    pltpu.make_async_copy(
        bufs[0], out_hbm.at[:, pl.ds((wid + 2 * NW) * CHUNK, CHUNK)], sems[0]
    ).wait()
    pltpu.make_async_copy(
        bufs[1], out_hbm.at[:, pl.ds((wid + NW) * CHUNK, CHUNK)], sems[1]
    ).wait()


DUP_CHUNK = 8192
N_DUP_BLOCKS = pl.cdiv(N_ATOMS, DUP_CHUNK)


def _onehot_block_tc(types_ref, o_ref):
    t = types_ref[:]
    rows = lax.broadcasted_iota(jnp.int32, (N_TYPES, t.shape[0]), 0)
    o_ref[...] = (rows == t[None, :]).astype(jnp.float32)


def _tail_tc(sc_ref, types_ref, o_ref):
    del sc_ref
    _onehot_block_tc(types_ref, o_ref)


_tail_call = pl.pallas_call(
    _tail_tc,
    grid=(1,),
    in_specs=[
        pl.BlockSpec(memory_space=pl.ANY),
        pl.BlockSpec((CHUNK,), lambda i: (N_BLOCKS - 1,)),
    ],
    out_specs=pl.BlockSpec((N_TYPES, CHUNK), lambda i: (0, N_BLOCKS - 1)),
    out_shape=jax.ShapeDtypeStruct((N_TYPES, N_ATOMS), jnp.float32),
    input_output_aliases={0: 0},
)

_dup_call = pl.pallas_call(
    _onehot_block_tc,
    grid=(N_DUP_BLOCKS,),
    in_specs=[pl.BlockSpec((DUP_CHUNK,), lambda i: (i,))],
    out_specs=pl.BlockSpec((N_TYPES, DUP_CHUNK), lambda i: (0, i)),
    out_shape=jax.ShapeDtypeStruct((N_TYPES, N_ATOMS), jnp.float32),
)


def kernel(atom_types, pos):
    del pos
    types = atom_types.reshape(-1)
    zeros = jnp.zeros((N_TYPES, CHUNK), jnp.float32)
    sc_out = _onehot_sc(types, zeros)
    out1 = _tail_call(sc_out, types)
    out2 = _dup_call(types)
    return (out1.T, out2.T)


# speedup vs baseline: 1.1943x; 1.1943x over previous
import functools

import jax
import jax.numpy as jnp
from jax import lax
from jax.experimental import pallas as pl
from jax.experimental.pallas import tpu as pltpu
from jax.experimental.pallas import tpu_sc as plsc

N_ATOMS = 100000
N_TYPES = 50
CHUNK = 512
SC_CHUNKS = 96
SC_COLS = SC_CHUNKS * CHUNK
GROUPS = CHUNK // 16
NW = 32

TC_BLOCK = 8192
TC_FIRST = SC_COLS // TC_BLOCK
N_TC_BLOCKS = pl.cdiv(N_ATOMS, TC_BLOCK) - TC_FIRST

DUP_BLOCK = 32768
N_DUP_BLOCKS = pl.cdiv(N_ATOMS, DUP_BLOCK)


@functools.partial(
    pl.kernel,
    out_type=jax.ShapeDtypeStruct((N_TYPES, N_ATOMS), jnp.float32),
    mesh=plsc.VectorSubcoreMesh(core_axis_name="c", subcore_axis_name="s"),
    scratch_types=[
        pltpu.VMEM((CHUNK,), jnp.int32),
        pltpu.VMEM((N_TYPES, CHUNK), jnp.float32),
    ],
    compiler_params=pltpu.CompilerParams(
        needs_layout_passes=False, skip_device_barrier=True
    ),
)
def _onehot_sc(types_hbm, zeros_hbm, out_hbm, types_v, buf):
    wid = lax.axis_index("s") * 2 + lax.axis_index("c")
    ones16 = jnp.ones((16,), jnp.float32)
    zeros16 = jnp.zeros((16,), jnp.float32)
    iota16 = lax.iota(jnp.int32, 16)

    pltpu.sync_copy(zeros_hbm, buf)

    for k in range(SC_CHUNKS // NW):
        c = wid + k * NW
        base = c * CHUNK
        pltpu.sync_copy(types_hbm.at[pl.ds(base, CHUNK)], types_v)
        for g in range(GROUPS):
            t = types_v[pl.ds(g * 16, 16)]
            plsc.store_scatter(buf, [t, iota16 + g * 16], ones16)
        pltpu.sync_copy(buf, out_hbm.at[:, pl.ds(base, CHUNK)])
        for g in range(GROUPS):
            t = types_v[pl.ds(g * 16, 16)]
            plsc.store_scatter(buf, [t, iota16 + g * 16], zeros16)


def _onehot_block_tc(types_ref, o_ref):
    t = types_ref[:]
    rows = lax.broadcasted_iota(jnp.int32, (N_TYPES, t.shape[0]), 0)
    o_ref[...] = (rows == t[None, :]).astype(jnp.float32)


def _region_tc(sc_ref, types_ref, o_ref):
    del sc_ref
    _onehot_block_tc(types_ref, o_ref)


_region_call = pl.pallas_call(
    _region_tc,
    grid=(N_TC_BLOCKS,),
    in_specs=[
        pl.BlockSpec(memory_space=pl.ANY),
        pl.BlockSpec((TC_BLOCK,), lambda i: (i + TC_FIRST,)),
    ],
    out_specs=pl.BlockSpec((N_TYPES, TC_BLOCK), lambda i: (0, i + TC_FIRST)),
    out_shape=jax.ShapeDtypeStruct((N_TYPES, N_ATOMS), jnp.float32),
    input_output_aliases={0: 0},
)

_dup_call = pl.pallas_call(
    _onehot_block_tc,
    grid=(N_DUP_BLOCKS,),
    in_specs=[pl.BlockSpec((DUP_BLOCK,), lambda i: (i,))],
    out_specs=pl.BlockSpec((N_TYPES, DUP_BLOCK), lambda i: (0, i)),
    out_shape=jax.ShapeDtypeStruct((N_TYPES, N_ATOMS), jnp.float32),
)


def kernel(atom_types, pos):
    del pos
    types = atom_types.reshape(-1)
    zeros = jnp.zeros((N_TYPES, CHUNK), jnp.float32)
    sc_out = _onehot_sc(types, zeros)
    out1 = _region_call(sc_out, types)
    out2 = _dup_call(types)
    return (out1.T, out2.T)


# speedup vs baseline: 1.5025x; 1.2580x over previous
import functools

import jax
import jax.numpy as jnp
from jax import lax
from jax.experimental import pallas as pl
from jax.experimental.pallas import tpu as pltpu
from jax.experimental.pallas import tpu_sc as plsc

N_ATOMS = 100000
N_TYPES = 50
CHUNK = 768
SC_CHUNKS = 64
SC_COLS = SC_CHUNKS * CHUNK
GROUPS = CHUNK // 16
NW = 32

TC_BLOCK = 16384
TC_FIRST = SC_COLS // TC_BLOCK
N_TC_BLOCKS = pl.cdiv(N_ATOMS, TC_BLOCK) - TC_FIRST

DUP_BLOCK = 50176
N_DUP_BLOCKS = pl.cdiv(N_ATOMS, DUP_BLOCK)


@functools.partial(
    pl.kernel,
    out_type=jax.ShapeDtypeStruct((N_TYPES, N_ATOMS), jnp.float32),
    mesh=plsc.VectorSubcoreMesh(core_axis_name="c", subcore_axis_name="s"),
    scratch_types=[
        pltpu.VMEM((CHUNK,), jnp.int32),
        pltpu.VMEM((N_TYPES, CHUNK), jnp.float32),
    ],
    compiler_params=pltpu.CompilerParams(
        needs_layout_passes=False, skip_device_barrier=True
    ),
)
def _onehot_sc(types_hbm, out_hbm, types_v, buf):
    wid = lax.axis_index("s") * 2 + lax.axis_index("c")
    ones16 = jnp.ones((16,), jnp.float32)
    zeros16 = jnp.zeros((16,), jnp.float32)
    iota16 = lax.iota(jnp.int32, 16)

    def zero_body(i, carry):
        for r in range(N_TYPES):
            buf[r, pl.ds(i * 16, 16)] = zeros16
        return carry

    lax.fori_loop(0, GROUPS, zero_body, 0)

    for k in range(SC_CHUNKS // NW):
        c = wid + k * NW
        base = c * CHUNK
        pltpu.sync_copy(types_hbm.at[pl.ds(base, CHUNK)], types_v)
        for g in range(GROUPS):
            t = types_v[pl.ds(g * 16, 16)]
            plsc.store_scatter(buf, [t, iota16 + g * 16], ones16)
        pltpu.sync_copy(buf, out_hbm.at[:, pl.ds(base, CHUNK)])
        if k + 1 < SC_CHUNKS // NW:
            for g in range(GROUPS):
                t = types_v[pl.ds(g * 16, 16)]
                plsc.store_scatter(buf, [t, iota16 + g * 16], zeros16)


def _onehot_block_tc(types_ref, o_ref):
    t = types_ref[:]
    rows = lax.broadcasted_iota(jnp.int32, (N_TYPES, t.shape[0]), 0)
    o_ref[...] = (rows == t[None, :]).astype(jnp.float32)


def _region_tc(sc_ref, types_ref, o_ref):
    del sc_ref
    _onehot_block_tc(types_ref, o_ref)


_region_call = pl.pallas_call(
    _region_tc,
    grid=(N_TC_BLOCKS,),
    in_specs=[
        pl.BlockSpec(memory_space=pl.ANY),
        pl.BlockSpec((TC_BLOCK,), lambda i: (i + TC_FIRST,)),
    ],
    out_specs=pl.BlockSpec((N_TYPES, TC_BLOCK), lambda i: (0, i + TC_FIRST)),
    out_shape=jax.ShapeDtypeStruct((N_TYPES, N_ATOMS), jnp.float32),
    input_output_aliases={0: 0},
)

_dup_call = pl.pallas_call(
    _onehot_block_tc,
    grid=(N_DUP_BLOCKS,),
    in_specs=[pl.BlockSpec((DUP_BLOCK,), lambda i: (i,))],
    out_specs=pl.BlockSpec((N_TYPES, DUP_BLOCK), lambda i: (0, i)),
    out_shape=jax.ShapeDtypeStruct((N_TYPES, N_ATOMS), jnp.float32),
)


def kernel(atom_types, pos):
    del pos
    types = atom_types.reshape(-1)
    sc_out = _onehot_sc(types)
    out1 = _region_call(sc_out, types)
    out2 = _dup_call(types)
    return (out1.T, out2.T)
